# Initial kernel scaffold; baseline (speedup 1.0000x reference)
import jax
import jax.numpy as jnp
from jax.experimental import pallas as pl


def kernel(x, edge_index, edge_vals, W1, b1, W2, b2, W3, b3):
    raise NotImplementedError("write your pallas kernel here")


# speedup vs baseline: 3.0114x; 3.0114x over previous
import functools

import jax
import jax.numpy as jnp
from jax import lax
from jax.experimental import pallas as pl
from jax.experimental.pallas import tpu as pltpu
from jax.experimental.pallas import tpu_sc as plsc

_N = 10000
_NP = 10240
_E = 320000

_NC = 2
_NS = 16
_NW = _NC * _NS
_EPW = _E // _NW
_CH = 80
_NCHUNK = _EPW // _CH
_RPT = _NP // _NS
_ZR = 32


def _lin_body(x_ref, w_ref, b_ref, o_ref):
    o_ref[...] = (
        jnp.dot(x_ref[...], w_ref[...], preferred_element_type=jnp.float32)
        + b_ref[...]
    )


def _mid_body(p0_ref, p1_ref, w_ref, b_ref, o_ref):
    h = jnp.maximum(p0_ref[...] + p1_ref[...], 0.0)
    o_ref[...] = (
        jnp.dot(h, w_ref[...], preferred_element_type=jnp.float32) + b_ref[...]
    )


def _argmax_body(p0_ref, p1_ref, o_ref):
    h = p0_ref[...] + p1_ref[...]
    col = lax.broadcasted_iota(jnp.int32, h.shape, 1)
    valid = col < 40
    h = jnp.where(valid, h, -jnp.inf)
    m = jnp.max(h, axis=1, keepdims=True)
    idx = jnp.min(jnp.where(h >= m, col, jnp.int32(2**30)), axis=1)
    o_ref[...] = idx[:, None]


_BR = 2048


def _bcast_body(e_ref, o_ref):
    o_ref[...] = jnp.broadcast_to(e_ref[...], o_ref.shape)


def _tc_ev_bcast(ev):
    e2 = ev.reshape(_E, 1)
    bre = 3200
    return pl.pallas_call(
        _bcast_body,
        grid=(_E // bre,),
        in_specs=[pl.BlockSpec((bre, 1), lambda i: (i, 0))],
        out_specs=pl.BlockSpec((bre, 16), lambda i: (i, 0)),
        out_shape=jax.ShapeDtypeStruct((_E, 16), jnp.float32),
    )(e2)


def _tc_first(x, W, b):
    n, k = x.shape
    m = W.shape[1]
    return pl.pallas_call(
        _lin_body,
        grid=(n // _BR,),
        in_specs=[
            pl.BlockSpec((_BR, k), lambda i: (i, 0)),
            pl.BlockSpec((k, m), lambda i: (0, 0)),
            pl.BlockSpec((1, m), lambda i: (0, 0)),
        ],
        out_specs=pl.BlockSpec((_BR, m), lambda i: (i, 0)),
        out_shape=jax.ShapeDtypeStruct((n, m), jnp.float32),
    )(x, W, b)


def _tc_mid(p0, p1, W, b):
    n, k = p0.shape
    m = W.shape[1]
    return pl.pallas_call(
        _mid_body,
        grid=(n // _BR,),
        in_specs=[
            pl.BlockSpec((_BR, k), lambda i: (i, 0)),
            pl.BlockSpec((_BR, k), lambda i: (i, 0)),
            pl.BlockSpec((k, m), lambda i: (0, 0)),
            pl.BlockSpec((1, m), lambda i: (0, 0)),
        ],
        out_specs=pl.BlockSpec((_BR, m), lambda i: (i, 0)),
        out_shape=jax.ShapeDtypeStruct((n, m), jnp.float32),
    )(p0, p1, W, b)


def _tc_argmax(p0, p1):
    n, k = p0.shape
    return pl.pallas_call(
        _argmax_body,
        grid=(n // _BR,),
        in_specs=[
            pl.BlockSpec((_BR, k), lambda i: (i, 0)),
            pl.BlockSpec((_BR, k), lambda i: (i, 0)),
        ],
        out_specs=pl.BlockSpec((_BR, 1), lambda i: (i, 0)),
        out_shape=jax.ShapeDtypeStruct((n, 1), jnp.int32),
    )(p0, p1)


def _spmm_sc_body(z_hbm, src_hbm, dst_hbm, ev_hbm, out_hbm,
                  idx_s, idx_d, evb, rows, zbuf, acc, sem):
    F = rows.shape[1]
    c = lax.axis_index("c")
    s = lax.axis_index("s")
    wid = c * _NS + s

    zero = jnp.zeros((16,), jnp.float32)
    for r in range(_ZR):
        for j in range(F // 16):
            zbuf[r, pl.ds(j * 16, 16)] = zero

    def zero_acc(i, carry):
        pltpu.sync_copy(zbuf, acc.at[pl.ds(s * _RPT + i * _ZR, _ZR)])
        return carry

    lax.fori_loop(0, _RPT // _ZR, zero_acc, 0)
    plsc.subcore_barrier()

    def chunk_body(k, carry):
        base = wid * _EPW + k * _CH
        pltpu.sync_copy(src_hbm.at[pl.ds(base, _CH)], idx_s)
        pltpu.sync_copy(dst_hbm.at[pl.ds(base, _CH)], idx_d)
        pltpu.sync_copy(ev_hbm.at[pl.ds(base, _CH)], evb)
        pltpu.async_copy(z_hbm.at[idx_s], rows, sem).wait()

        def scale_body(e, c2):
            evx = evb[e, pl.ds(0, 16)]
            for j in range(F // 16):
                sl = pl.ds(j * 16, 16)
                rows[e, sl] = rows[e, sl] * evx
            return c2

        lax.fori_loop(0, _CH, scale_body, 0)
        pltpu.sync_copy(rows, acc.at[idx_d], add=True)
        return carry

    lax.fori_loop(0, _NCHUNK, chunk_body, 0)
    plsc.subcore_barrier()

    pltpu.sync_copy(acc.at[pl.ds(s * _RPT, _RPT)],
                    out_hbm.at[c, pl.ds(s * _RPT, _RPT)])


def _spmm(z, src, dst, ev):
    n, F = z.shape
    mesh = plsc.VectorSubcoreMesh(core_axis_name="c", subcore_axis_name="s")
    f = pl.kernel(
        _spmm_sc_body,
        out_type=jax.ShapeDtypeStruct((_NC, n, F), jnp.float32),
        mesh=mesh,
        scratch_types=[
            pltpu.VMEM((_CH,), jnp.int32),
            pltpu.VMEM((_CH,), jnp.int32),
            pltpu.VMEM((_CH, 16), jnp.float32),
            pltpu.VMEM((_CH, F), jnp.float32),
            pltpu.VMEM((_ZR, F), jnp.float32),
            pltpu.VMEM_SHARED((n, F), jnp.float32),
            pltpu.SemaphoreType.DMA,
        ],
        compiler_params=pltpu.CompilerParams(use_tc_tiling_on_sc=False),
    )
    return f(z, src, dst, ev)


def kernel(x, edge_index, edge_vals, W1, b1, W2, b2, W3, b3):
    dst = edge_index[0]
    src = edge_index[1]

    xp = jnp.pad(x, ((0, _NP - _N), (0, 0)))
    z1 = _tc_first(xp, W1, b1.reshape(1, -1))
    evb = _tc_ev_bcast(edge_vals)
    p1 = _spmm(z1, src, dst, evb)
    z2 = _tc_mid(p1[0], p1[1], W2, b2.reshape(1, -1))
    p2 = _spmm(z2, src, dst, evb)
    W3p = jnp.pad(W3, ((0, 0), (0, 8)))
    b3p = jnp.pad(b3, (0, 8)).reshape(1, -1)
    z3 = _tc_mid(p2[0], p2[1], W3p, b3p)
    p3 = _spmm(z3, src, dst, evb)
    out = _tc_argmax(p3[0], p3[1])
    return out[:_N, 0]


# speedup vs baseline: 3.0482x; 1.0122x over previous
import functools

import jax
import jax.numpy as jnp
from jax import lax
from jax.experimental import pallas as pl
from jax.experimental.pallas import tpu as pltpu
from jax.experimental.pallas import tpu_sc as plsc

_N = 10000
_NP = 10240
_E = 320000
_EPAD = 327680

_NC = 2
_NS = 16
_NW = _NC * _NS
_EPW = _EPAD // _NW
_CH = 128
_NCHUNK = _EPW // _CH
_RPT = _NP // _NS
_ZR = 32


def _lin_body(x_ref, w_ref, b_ref, o_ref):
    o_ref[...] = (
        jnp.dot(x_ref[...], w_ref[...], preferred_element_type=jnp.float32)
        + b_ref[...]
    )


def _mid_body(p0_ref, p1_ref, w_ref, b_ref, o_ref):
    h = jnp.maximum(p0_ref[...] + p1_ref[...], 0.0)
    o_ref[...] = (
        jnp.dot(h, w_ref[...], preferred_element_type=jnp.float32) + b_ref[...]
    )


def _argmax_body(p0_ref, p1_ref, o_ref):
    h = p0_ref[...] + p1_ref[...]
    col = lax.broadcasted_iota(jnp.int32, h.shape, 1)
    valid = col < 40
    h = jnp.where(valid, h, -jnp.inf)
    m = jnp.max(h, axis=1, keepdims=True)
    idx = jnp.min(jnp.where(h >= m, col, jnp.int32(2**30)), axis=1)
    o_ref[...] = idx[:, None]


_BR = 2048


def _bcast_body(e_ref, o_ref):
    o_ref[...] = jnp.broadcast_to(e_ref[...], o_ref.shape)


def _tc_ev_bcast(ev):
    e2 = ev.reshape(_EPAD, 1)
    bre = 4096
    return pl.pallas_call(
        _bcast_body,
        grid=(_EPAD // bre,),
        in_specs=[pl.BlockSpec((bre, 1), lambda i: (i, 0))],
        out_specs=pl.BlockSpec((bre, 16), lambda i: (i, 0)),
        out_shape=jax.ShapeDtypeStruct((_EPAD, 16), jnp.float32),
    )(e2)


def _tc_first(x, W, b):
    n, k = x.shape
    m = W.shape[1]
    return pl.pallas_call(
        _lin_body,
        grid=(n // _BR,),
        in_specs=[
            pl.BlockSpec((_BR, k), lambda i: (i, 0)),
            pl.BlockSpec((k, m), lambda i: (0, 0)),
            pl.BlockSpec((1, m), lambda i: (0, 0)),
        ],
        out_specs=pl.BlockSpec((_BR, m), lambda i: (i, 0)),
        out_shape=jax.ShapeDtypeStruct((n, m), jnp.float32),
    )(x, W, b)


def _tc_mid(p0, p1, W, b):
    n, k = p0.shape
    m = W.shape[1]
    return pl.pallas_call(
        _mid_body,
        grid=(n // _BR,),
        in_specs=[
            pl.BlockSpec((_BR, k), lambda i: (i, 0)),
            pl.BlockSpec((_BR, k), lambda i: (i, 0)),
            pl.BlockSpec((k, m), lambda i: (0, 0)),
            pl.BlockSpec((1, m), lambda i: (0, 0)),
        ],
        out_specs=pl.BlockSpec((_BR, m), lambda i: (i, 0)),
        out_shape=jax.ShapeDtypeStruct((n, m), jnp.float32),
    )(p0, p1, W, b)


def _tc_argmax(p0, p1):
    n, k = p0.shape
    return pl.pallas_call(
        _argmax_body,
        grid=(n // _BR,),
        in_specs=[
            pl.BlockSpec((_BR, k), lambda i: (i, 0)),
            pl.BlockSpec((_BR, k), lambda i: (i, 0)),
        ],
        out_specs=pl.BlockSpec((_BR, 1), lambda i: (i, 0)),
        out_shape=jax.ShapeDtypeStruct((n, 1), jnp.int32),
    )(p0, p1)


def _spmm_sc_body(z_hbm, src_hbm, dst_hbm, ev_hbm, out_hbm,
                  src_all, dst_all, evb, rows, zbuf, acc, sem):
    F = rows.shape[1]
    c = lax.axis_index("c")
    s = lax.axis_index("s")
    wid = c * _NS + s

    pltpu.sync_copy(src_hbm.at[pl.ds(wid * _NCHUNK, _NCHUNK)], src_all)
    pltpu.sync_copy(dst_hbm.at[pl.ds(wid * _NCHUNK, _NCHUNK)], dst_all)

    zero = jnp.zeros((16,), jnp.float32)
    for r in range(_ZR):
        for j in range(F // 16):
            zbuf[r, pl.ds(j * 16, 16)] = zero

    def zero_acc(i, carry):
        pltpu.sync_copy(zbuf, acc.at[pl.ds(s * _RPT + i * _ZR, _ZR)])
        return carry

    lax.fori_loop(0, _RPT // _ZR, zero_acc, 0)
    plsc.subcore_barrier()

    def chunk_body(k, carry):
        row = wid * _NCHUNK + k
        c_ev = pltpu.async_copy(ev_hbm.at[row], evb, sem)
        c_g = pltpu.async_copy(z_hbm.at[src_all.at[k]], rows, sem)
        c_ev.wait()
        c_g.wait()

        def scale_body(e, c2):
            evx = evb[e, pl.ds(0, 16)]
            for j in range(F // 16):
                sl = pl.ds(j * 16, 16)
                rows[e, sl] = rows[e, sl] * evx
            return c2

        lax.fori_loop(0, _CH, scale_body, 0)
        pltpu.sync_copy(rows, acc.at[dst_all.at[k]], add=True)
        return carry

    lax.fori_loop(0, _NCHUNK, chunk_body, 0)
    plsc.subcore_barrier()

    pltpu.sync_copy(acc.at[pl.ds(s * _RPT, _RPT)],
                    out_hbm.at[c, pl.ds(s * _RPT, _RPT)])


def _spmm(z, src, dst, ev):
    n, F = z.shape
    mesh = plsc.VectorSubcoreMesh(core_axis_name="c", subcore_axis_name="s")
    f = pl.kernel(
        _spmm_sc_body,
        out_type=jax.ShapeDtypeStruct((_NC, n, F), jnp.float32),
        mesh=mesh,
        scratch_types=[
            pltpu.VMEM((_NCHUNK, _CH), jnp.int32),
            pltpu.VMEM((_NCHUNK, _CH), jnp.int32),
            pltpu.VMEM((_CH, 16), jnp.float32),
            pltpu.VMEM((_CH, F), jnp.float32),
            pltpu.VMEM((_ZR, F), jnp.float32),
            pltpu.VMEM_SHARED((n, F), jnp.float32),
            pltpu.SemaphoreType.DMA,
        ],
        compiler_params=pltpu.CompilerParams(use_tc_tiling_on_sc=False),
    )
    return f(z, src, dst, ev)


def kernel(x, edge_index, edge_vals, W1, b1, W2, b2, W3, b3):
    pad_e = _EPAD - _E
    dst = jnp.pad(edge_index[0], (0, pad_e)).reshape(_NW * _NCHUNK, _CH)
    src = jnp.pad(edge_index[1], (0, pad_e)).reshape(_NW * _NCHUNK, _CH)
    evp = jnp.pad(edge_vals, (0, pad_e))

    xp = jnp.pad(x, ((0, _NP - _N), (0, 0)))
    z1 = _tc_first(xp, W1, b1.reshape(1, -1))
    evb = _tc_ev_bcast(evp).reshape(_NW * _NCHUNK, _CH, 16)
    p1 = _spmm(z1, src, dst, evb)
    z2 = _tc_mid(p1[0], p1[1], W2, b2.reshape(1, -1))
    p2 = _spmm(z2, src, dst, evb)
    W3p = jnp.pad(W3, ((0, 0), (0, 8)))
    b3p = jnp.pad(b3, (0, 8)).reshape(1, -1)
    z3 = _tc_mid(p2[0], p2[1], W3p, b3p)
    p3 = _spmm(z3, src, dst, evb)
    out = _tc_argmax(p3[0], p3[1])
    return out[:_N, 0]
